# baseline (speedup 1.0000x reference)
import functools

import jax
import jax.numpy as jnp
from jax import lax
from jax.experimental import pallas as pl
from jax.experimental.pallas import tpu as pltpu
from jax.experimental.pallas import tpu_sc as plsc

_NROWS = 16384
_NIN = 3328
_NOUT = 2048
_W0 = 1024
_S1 = 2304
_W1 = 1024

_NC = 2
_NS = 16
_NW = _NC * _NS
_RPW = _NROWS // _NW
_R = 16
_NB = 2
_NCHUNK = _RPW // _R
_NPAIR = _NCHUNK // 2


@functools.partial(
    pl.kernel,
    mesh=plsc.VectorSubcoreMesh(core_axis_name="c", subcore_axis_name="s"),
    out_type=jax.ShapeDtypeStruct((_NROWS, _NOUT), jnp.float32),
    scratch_types=(
        [pltpu.VMEM((_NB, _R, _NOUT), jnp.float32),
         pltpu.VMEM_SHARED((_NS, _NB, _R, _NOUT), jnp.float32)]
        + [pltpu.SemaphoreType.DMA] * (4 * _NB)
    ),
)
def _route(feat, out, vbuf, shbuf, *sems):
    cid = lax.axis_index("c")
    sid = lax.axis_index("s")
    wid = sid * _NC + cid
    base = wid * _RPW
    vsin = sems[0:_NB]
    vsout = sems[_NB:2 * _NB]
    ssin = sems[2 * _NB:3 * _NB]
    ssout = sems[3 * _NB:4 * _NB]

    def mk(i, b, ring):
        r0 = base + i * _R
        if ring == 0:
            dst = vbuf.at[b]
            sin, sout = vsin[b], vsout[b]
        else:
            dst = shbuf.at[sid, b]
            sin, sout = ssin[b], ssout[b]
        ca = pltpu.make_async_copy(
            feat.at[pl.ds(r0, _R), pl.ds(0, _W0)], dst.at[:, pl.ds(0, _W0)],
            sin)
        cb = pltpu.make_async_copy(
            feat.at[pl.ds(r0, _R), pl.ds(_S1, _W1)], dst.at[:, pl.ds(_W0, _W1)],
            sin)
        co = pltpu.make_async_copy(dst, out.at[pl.ds(r0, _R)], sout)
        return ca, cb, co

    def chunk_of(g, ring):
        return 2 * g + ring

    for g0 in range(_NB):
        for ring in range(2):
            ca, cb, _ = mk(chunk_of(g0, ring), g0, ring)
            ca.start()
            cb.start()

    def grp(g, carry):
        for b in range(_NB):
            p = g * _NB + b
            cavs = [mk(chunk_of(p, ring), b, ring) for ring in range(2)]
            for ca, cb, _ in cavs:
                ca.wait()
                cb.wait()
            for _, _, co in cavs:
                co.start()
            jb = (b - 1) % _NB

            @pl.when((p >= 1) & (p - 1 + _NB < _NPAIR))
            def _prefetch():
                for ring in range(2):
                    _, _, po = mk(chunk_of(p - 1, ring), jb, ring)
                    po.wait()
                    na, nb_, _ = mk(chunk_of(p - 1 + _NB, ring), jb, ring)
                    na.start()
                    nb_.start()
        return carry

    lax.fori_loop(0, _NPAIR // _NB, grp, 0)

    for b in range(_NB):
        for ring in range(2):
            _, _, co = mk(chunk_of(_NPAIR - _NB + b, ring), b, ring)
            co.wait()


def kernel(features, idx):
    del idx
    return _route(features)


# speedup vs baseline: 1.0048x; 1.0048x over previous
import functools

import jax
import jax.numpy as jnp
from jax import lax
from jax.experimental import pallas as pl
from jax.experimental.pallas import tpu as pltpu
from jax.experimental.pallas import tpu_sc as plsc

_NROWS = 16384
_NIN = 3328
_NOUT = 2048
_W0 = 1024
_S1 = 2304
_W1 = 1024

_NC = 2
_NS = 16
_NW = _NC * _NS
_RPW = _NROWS // _NW
_R = 16
_NB = 2
_NCHUNK = _RPW // _R
_NPAIR = _NCHUNK // 2


@functools.partial(
    pl.kernel,
    mesh=plsc.VectorSubcoreMesh(core_axis_name="c", subcore_axis_name="s"),
    out_type=jax.ShapeDtypeStruct((_NROWS, _NOUT), jnp.float32),
    scratch_types=(
        [pltpu.VMEM((_NB, _R, _NOUT), jnp.float32),
         pltpu.VMEM_SHARED((_NS, _NB, _R, _NOUT), jnp.float32)]
        + [pltpu.SemaphoreType.DMA] * (4 * _NB)
    ),
)
def _route(feat, out, vbuf, shbuf, *sems):
    cid = lax.axis_index("c")
    sid = lax.axis_index("s")
    wid = sid * _NC + cid
    base = wid * _RPW
    vsin = sems[0:_NB]
    vsout = sems[_NB:2 * _NB]
    ssin = sems[2 * _NB:3 * _NB]
    ssout = sems[3 * _NB:4 * _NB]

    def mk(i, b, ring):
        r0 = base + i * _R
        if ring == 0:
            dst = vbuf.at[b]
            sin, sout = vsin[b], vsout[b]
        else:
            dst = shbuf.at[sid, b]
            sin, sout = ssin[b], ssout[b]
        ca = pltpu.make_async_copy(
            feat.at[pl.ds(r0, _R), pl.ds(0, _W0)], dst.at[:, pl.ds(0, _W0)],
            sin)
        cb = pltpu.make_async_copy(
            feat.at[pl.ds(r0, _R), pl.ds(_S1, _W1)], dst.at[:, pl.ds(_W0, _W1)],
            sin)
        co = pltpu.make_async_copy(dst, out.at[pl.ds(r0, _R)], sout)
        return ca, cb, co

    def chunk_of(g, ring):
        return 2 * g + ring

    for g0 in range(_NB):
        for ring in range(2):
            ca, cb, _ = mk(chunk_of(g0, ring), g0, ring)
            ca.start()
            cb.start()

    def grp(g, carry):
        for b in range(_NB):
            p = g * _NB + b
            for ring in range(2):
                ca, cb, co = mk(chunk_of(p, ring), b, ring)
                ca.wait()
                cb.wait()
                co.start()
            jb = (b - 1) % _NB

            @pl.when((p >= 1) & (p - 1 + _NB < _NPAIR))
            def _prefetch():
                for ring in range(2):
                    _, _, po = mk(chunk_of(p - 1, ring), jb, ring)
                    po.wait()
                    na, nb_, _ = mk(chunk_of(p - 1 + _NB, ring), jb, ring)
                    na.start()
                    nb_.start()
        return carry

    lax.fori_loop(0, _NPAIR // _NB, grp, 0)

    for b in range(_NB):
        for ring in range(2):
            _, _, co = mk(chunk_of(_NPAIR - _NB + b, ring), b, ring)
            co.wait()


def kernel(features, idx):
    del idx
    return _route(features)
